# Initial kernel scaffold; baseline (speedup 1.0000x reference)
import jax
import jax.numpy as jnp
from jax.experimental import pallas as pl


def kernel(x, edge_index, W1_l, b1, W1_r, W2_l, b2, W2_r, W3_l, b3, W3_r):
    raise NotImplementedError("write your pallas kernel here")


# speedup vs baseline: 4.1135x; 4.1135x over previous
import functools

import jax
import jax.numpy as jnp
from jax import lax
from jax.experimental import pallas as pl
from jax.experimental.pallas import tpu as pltpu
from jax.experimental.pallas import tpu_sc as plsc

N = 10000
E = 160000
N_PAD = 10240
LANE = 128
NC, NS = 2, 16
EPT = E // NS
CHUNK = 80
CHUNKS = EPT // CHUNK
ZR = N_PAD // NS


@functools.cache
def _segsum_sc(P):
    split = (P + 1) // 2
    mesh = plsc.VectorSubcoreMesh(core_axis_name="c", subcore_axis_name="s",
                                  num_cores=NC, num_subcores=NS)

    @functools.partial(
        pl.kernel,
        out_type=jax.ShapeDtypeStruct((P * N_PAD, LANE), jnp.float32),
        mesh=mesh,
        scratch_types=[
            pltpu.VMEM((EPT,), jnp.int32),
            pltpu.VMEM((CHUNKS, CHUNK), jnp.int32),
            pltpu.VMEM((CHUNK,), jnp.int32),
            pltpu.VMEM((CHUNK, LANE), jnp.float32),
            pltpu.VMEM_SHARED((N_PAD, LANE), jnp.float32),
            pltpu.SemaphoreType.DMA,
        ],
    )
    def k(table, src, dst, zeros, out, srcb, dstb, gidx, rows, acc, sem):
        c = lax.axis_index("c")
        s = lax.axis_index("s")
        pltpu.sync_copy(src.at[s], srcb)
        pltpu.sync_copy(dst.at[s], dstb)
        my_base = c * split
        my_count = jnp.where(c == 0, split, P - split)
        for pp in range(split):
            @pl.when(pp < my_count)
            def _():
                off = (my_base + pp) * N_PAD
                pltpu.sync_copy(zeros, acc.at[pl.ds(s * ZR, ZR)])
                plsc.subcore_barrier()

                def body(j, _):
                    for i in range(CHUNK // 16):
                        v = srcb[pl.ds(j * CHUNK + i * 16, 16)]
                        gidx[pl.ds(i * 16, 16)] = v + off
                    pltpu.async_copy(table.at[gidx], rows, sem).wait()
                    pltpu.sync_copy(rows, acc.at[dstb.at[j]], add=True)
                    return ()

                lax.fori_loop(0, CHUNKS, body, (), unroll=False)
                plsc.subcore_barrier()
                pltpu.sync_copy(acc.at[pl.ds(s * ZR, ZR)],
                                out.at[pl.ds(off + s * ZR, ZR)])
                plsc.subcore_barrier()

    return k


def _tc_layer(P_in, D_out, BN=256, JB=512, interpret=False):
    JP = JB // LANE
    OP = D_out // LANE
    K = P_in * LANE

    def body(agg_ref, deg_ref, x_ref, wl_ref, wr_ref, b_ref, out_ref):
        invd = 1.0 / jnp.maximum(deg_ref[...], 1.0)
        a = jnp.concatenate([agg_ref[p] for p in range(P_in)], axis=1)
        xx = jnp.concatenate([x_ref[p] for p in range(P_in)], axis=1)
        acc = jnp.dot(a, wl_ref[...], preferred_element_type=jnp.float32) * invd
        acc = acc + jnp.dot(xx, wr_ref[...], preferred_element_type=jnp.float32)
        acc = jnp.maximum(acc + b_ref[...], 0.0)
        for q in range(JP):
            out_ref[q] = acc[:, q * LANE:(q + 1) * LANE]

    return pl.pallas_call(
        body,
        grid=(D_out // JB, N_PAD // BN),
        in_specs=[
            pl.BlockSpec((P_in, BN, LANE), lambda j, n: (0, n, 0)),
            pl.BlockSpec((BN, 1), lambda j, n: (n, 0)),
            pl.BlockSpec((P_in, BN, LANE), lambda j, n: (0, n, 0)),
            pl.BlockSpec((K, JB), lambda j, n: (0, j)),
            pl.BlockSpec((K, JB), lambda j, n: (0, j)),
            pl.BlockSpec((1, JB), lambda j, n: (0, j)),
        ],
        out_specs=pl.BlockSpec((JP, BN, LANE), lambda j, n: (j, n, 0)),
        out_shape=jax.ShapeDtypeStruct((OP, N_PAD, LANE), jnp.float32),
        interpret=interpret,
    )


def _tc_matmul(P_in, D_out, BN=256, interpret=False):
    OP = D_out // LANE
    K = P_in * LANE

    def body(x_ref, w_ref, out_ref):
        xx = jnp.concatenate([x_ref[p] for p in range(P_in)], axis=1)
        acc = jnp.dot(xx, w_ref[...], preferred_element_type=jnp.float32)
        for q in range(OP):
            out_ref[q] = acc[:, q * LANE:(q + 1) * LANE]

    return pl.pallas_call(
        body,
        grid=(N_PAD // BN,),
        in_specs=[
            pl.BlockSpec((P_in, BN, LANE), lambda n: (0, n, 0)),
            pl.BlockSpec((K, D_out), lambda n: (0, 0)),
        ],
        out_specs=pl.BlockSpec((OP, BN, LANE), lambda n: (0, n, 0)),
        out_shape=jax.ShapeDtypeStruct((OP, N_PAD, LANE), jnp.float32),
        interpret=interpret,
    )


def _tc_final(P_in, D_out, BN=256, interpret=False):
    AP = D_out // LANE
    K = P_in * LANE

    def body(agg_ref, deg_ref, x_ref, wr_ref, b_ref, out_ref):
        invd = 1.0 / jnp.maximum(deg_ref[...], 1.0)
        xx = jnp.concatenate([x_ref[p] for p in range(P_in)], axis=1)
        acc = jnp.dot(xx, wr_ref[...], preferred_element_type=jnp.float32)
        agg = jnp.concatenate([agg_ref[q] for q in range(AP)], axis=1)
        out_ref[...] = jnp.maximum(acc + agg * invd + b_ref[...], 0.0)

    return pl.pallas_call(
        body,
        grid=(N_PAD // BN,),
        in_specs=[
            pl.BlockSpec((AP, BN, LANE), lambda n: (0, n, 0)),
            pl.BlockSpec((BN, 1), lambda n: (n, 0)),
            pl.BlockSpec((P_in, BN, LANE), lambda n: (0, n, 0)),
            pl.BlockSpec((K, D_out), lambda n: (0, 0)),
            pl.BlockSpec((1, D_out), lambda n: (0, 0)),
        ],
        out_specs=pl.BlockSpec((BN, D_out), lambda n: (n, 0)),
        out_shape=jax.ShapeDtypeStruct((N_PAD, D_out), jnp.float32),
        interpret=interpret,
    )


_l1 = _tc_layer(2, 1024)
_l2 = _tc_layer(8, 1024)
_mm3 = _tc_matmul(8, 256)
_fin = _tc_final(8, 256)


def _panel_major(h, P):
    hp = jnp.pad(h, ((0, N_PAD - h.shape[0]), (0, 0)))
    return hp.reshape(N_PAD, P, LANE).transpose(1, 0, 2).reshape(P * N_PAD, LANE)


def kernel(x, edge_index, W1_l, b1, W1_r, W2_l, b2, W2_r, W3_l, b3, W3_r):
    src = edge_index[0].reshape(NS, EPT)
    dst = edge_index[1].reshape(NS, CHUNKS, CHUNK)
    zeros = jnp.zeros((ZR, LANE), jnp.float32)

    x_pm = _panel_major(x, 2)
    table1 = jnp.concatenate([x_pm, jnp.ones((N_PAD, LANE), jnp.float32)], 0)
    agg1 = _segsum_sc(3)(table1, src, dst, zeros)
    deg = agg1[2 * N_PAD:2 * N_PAD + N_PAD, 0:1]
    h1 = _l1(agg1[:2 * N_PAD].reshape(2, N_PAD, LANE), deg,
             x_pm.reshape(2, N_PAD, LANE), W1_l, W1_r, b1.reshape(1, -1))

    agg2 = _segsum_sc(8)(h1.reshape(8 * N_PAD, LANE), src, dst, zeros)
    h2 = _l2(agg2.reshape(8, N_PAD, LANE), deg, h1,
             W2_l, W2_r, b2.reshape(1, -1))

    p3 = _mm3(h2, W3_l)
    agg3 = _segsum_sc(2)(p3.reshape(2 * N_PAD, LANE), src, dst, zeros)
    out = _fin(agg3.reshape(2, N_PAD, LANE), deg, h2, W3_r, b3.reshape(1, -1))
    return out[:N]


# speedup vs baseline: 5.5024x; 1.3376x over previous
import functools

import jax
import jax.numpy as jnp
from jax import lax
from jax.experimental import pallas as pl
from jax.experimental.pallas import tpu as pltpu
from jax.experimental.pallas import tpu_sc as plsc

N = 10000
E = 160000
N_PAD = 10240
LANE = 128
NC, NS = 2, 16
EPT = E // NS
CHUNK = 80
CHUNKS = EPT // CHUNK
ZR = N_PAD // NS


def _sc_mesh():
    return plsc.VectorSubcoreMesh(core_axis_name="c", subcore_axis_name="s",
                                  num_cores=NC, num_subcores=NS)


@functools.cache
def _segsum_sc(P):
    split = (P + 1) // 2

    @functools.partial(
        pl.kernel,
        out_type=jax.ShapeDtypeStruct((P * N_PAD, LANE), jnp.float32),
        mesh=_sc_mesh(),
        scratch_types=[
            pltpu.VMEM((EPT,), jnp.int32),
            pltpu.VMEM((CHUNKS, CHUNK), jnp.int32),
            pltpu.VMEM((CHUNK,), jnp.int32),
            pltpu.VMEM((CHUNK,), jnp.int32),
            pltpu.VMEM((CHUNK, LANE), jnp.float32),
            pltpu.VMEM((CHUNK, LANE), jnp.float32),
            pltpu.VMEM_SHARED((N_PAD, LANE), jnp.float32),
            pltpu.SemaphoreType.DMA,
            pltpu.SemaphoreType.DMA,
        ],
    )
    def k(table, src, dst, zeros, out, srcb, dstb, g0, g1, r0, r1, acc,
          gs0, gs1):
        c = lax.axis_index("c")
        s = lax.axis_index("s")
        pltpu.sync_copy(src.at[s], srcb)
        pltpu.sync_copy(dst.at[s], dstb)
        my_base = c * split
        my_count = jnp.where(c == 0, split, P - split)
        for pp in range(split):
            @pl.when(pp < my_count)
            def _():
                off = (my_base + pp) * N_PAD

                def build(jv, gb):
                    for i in range(CHUNK // 16):
                        v = srcb[pl.ds(jv * CHUNK + i * 16, 16)]
                        gb[pl.ds(i * 16, 16)] = v + off

                pltpu.sync_copy(zeros, acc.at[pl.ds(s * ZR, ZR)])
                plsc.subcore_barrier()
                build(0, g0)
                pltpu.async_copy(table.at[g0], r0, gs0)

                def pair(t, _):
                    j = 2 * t
                    pltpu.make_async_copy(table.at[g0], r0, gs0).wait()
                    build(j + 1, g1)
                    pltpu.async_copy(table.at[g1], r1, gs1)
                    pltpu.sync_copy(r0, acc.at[dstb.at[j]], add=True)
                    pltpu.make_async_copy(table.at[g1], r1, gs1).wait()
                    build(j + 2, g0)
                    pltpu.async_copy(table.at[g0], r0, gs0)
                    pltpu.sync_copy(r1, acc.at[dstb.at[j + 1]], add=True)
                    return ()

                lax.fori_loop(0, (CHUNKS - 1) // 2, pair, (), unroll=False)
                pltpu.make_async_copy(table.at[g0], r0, gs0).wait()
                pltpu.sync_copy(r0, acc.at[dstb.at[CHUNKS - 1]], add=True)
                plsc.subcore_barrier()
                pltpu.sync_copy(acc.at[pl.ds(s * ZR, ZR)],
                                out.at[pl.ds(off + s * ZR, ZR)])
                plsc.subcore_barrier()

    return lambda *a: k(*a)


@functools.cache
def _deg_sc():
    @functools.partial(
        pl.kernel,
        out_type=jax.ShapeDtypeStruct((N_PAD, LANE), jnp.float32),
        mesh=_sc_mesh(),
        scratch_types=[
            pltpu.VMEM((CHUNKS, CHUNK), jnp.int32),
            pltpu.VMEM((CHUNK, LANE), jnp.float32),
            pltpu.VMEM_SHARED((N_PAD, LANE), jnp.float32),
            pltpu.SemaphoreType.DMA,
        ],
    )
    def k(dst, zeros, out, dstb, onesb, dacc, sem):
        c = lax.axis_index("c")
        s = lax.axis_index("s")

        @pl.when(c == 0)
        def _():
            pltpu.sync_copy(dst.at[s], dstb)
            ones16 = jnp.ones((16,), jnp.float32)
            for r in range(CHUNK):
                for w in range(LANE // 16):
                    onesb[r, pl.ds(w * 16, 16)] = ones16
            pltpu.sync_copy(zeros, dacc.at[pl.ds(s * ZR, ZR)])
            plsc.subcore_barrier()

            def fire(j, _):
                pltpu.sync_copy(onesb, dacc.at[dstb.at[j]], add=True)
                return ()

            lax.fori_loop(0, CHUNKS, fire, (), unroll=False)
            plsc.subcore_barrier()
            pltpu.sync_copy(dacc.at[pl.ds(s * ZR, ZR)],
                            out.at[pl.ds(s * ZR, ZR)])

    return k


def _tc_layer(P_in, D_out, BN=256, JB=512, interpret=False):
    JP = JB // LANE
    OP = D_out // LANE
    K = P_in * LANE

    def body(agg_ref, deg_ref, x_ref, wl_ref, wr_ref, b_ref, out_ref):
        invd = 1.0 / jnp.maximum(deg_ref[...], 1.0)
        a = jnp.concatenate([agg_ref[p] for p in range(P_in)], axis=1)
        xx = jnp.concatenate([x_ref[p] for p in range(P_in)], axis=1)
        acc = jnp.dot(a, wl_ref[...], preferred_element_type=jnp.float32) * invd
        acc = acc + jnp.dot(xx, wr_ref[...], preferred_element_type=jnp.float32)
        acc = jnp.maximum(acc + b_ref[...], 0.0)
        for q in range(JP):
            out_ref[q] = acc[:, q * LANE:(q + 1) * LANE]

    return pl.pallas_call(
        body,
        grid=(D_out // JB, N_PAD // BN),
        in_specs=[
            pl.BlockSpec((P_in, BN, LANE), lambda j, n: (0, n, 0)),
            pl.BlockSpec((BN, 1), lambda j, n: (n, 0)),
            pl.BlockSpec((P_in, BN, LANE), lambda j, n: (0, n, 0)),
            pl.BlockSpec((K, JB), lambda j, n: (0, j)),
            pl.BlockSpec((K, JB), lambda j, n: (0, j)),
            pl.BlockSpec((1, JB), lambda j, n: (0, j)),
        ],
        out_specs=pl.BlockSpec((JP, BN, LANE), lambda j, n: (j, n, 0)),
        out_shape=jax.ShapeDtypeStruct((OP, N_PAD, LANE), jnp.float32),
        interpret=interpret,
    )


def _tc_matmul(P_in, D_out, BN=256, interpret=False):
    OP = D_out // LANE
    K = P_in * LANE

    def body(x_ref, w_ref, out_ref):
        xx = jnp.concatenate([x_ref[p] for p in range(P_in)], axis=1)
        acc = jnp.dot(xx, w_ref[...], preferred_element_type=jnp.float32)
        for q in range(OP):
            out_ref[q] = acc[:, q * LANE:(q + 1) * LANE]

    return pl.pallas_call(
        body,
        grid=(N_PAD // BN,),
        in_specs=[
            pl.BlockSpec((P_in, BN, LANE), lambda n: (0, n, 0)),
            pl.BlockSpec((K, D_out), lambda n: (0, 0)),
        ],
        out_specs=pl.BlockSpec((OP, BN, LANE), lambda n: (0, n, 0)),
        out_shape=jax.ShapeDtypeStruct((OP, N_PAD, LANE), jnp.float32),
        interpret=interpret,
    )


def _tc_final(P_in, D_out, BN=256, interpret=False):
    AP = D_out // LANE
    K = P_in * LANE

    def body(agg_ref, deg_ref, x_ref, wr_ref, b_ref, out_ref):
        invd = 1.0 / jnp.maximum(deg_ref[...], 1.0)
        xx = jnp.concatenate([x_ref[p] for p in range(P_in)], axis=1)
        acc = jnp.dot(xx, wr_ref[...], preferred_element_type=jnp.float32)
        agg = jnp.concatenate([agg_ref[q] for q in range(AP)], axis=1)
        out_ref[...] = jnp.maximum(acc + agg * invd + b_ref[...], 0.0)

    return pl.pallas_call(
        body,
        grid=(N_PAD // BN,),
        in_specs=[
            pl.BlockSpec((AP, BN, LANE), lambda n: (0, n, 0)),
            pl.BlockSpec((BN, 1), lambda n: (n, 0)),
            pl.BlockSpec((P_in, BN, LANE), lambda n: (0, n, 0)),
            pl.BlockSpec((K, D_out), lambda n: (0, 0)),
            pl.BlockSpec((1, D_out), lambda n: (0, 0)),
        ],
        out_specs=pl.BlockSpec((BN, D_out), lambda n: (n, 0)),
        out_shape=jax.ShapeDtypeStruct((N_PAD, D_out), jnp.float32),
        interpret=interpret,
    )


_l1 = _tc_layer(2, 1024)
_l2 = _tc_layer(8, 1024)
_mm3 = _tc_matmul(8, 256)
_fin = _tc_final(8, 256)


def _panel_major(h, P):
    hp = jnp.pad(h, ((0, N_PAD - h.shape[0]), (0, 0)))
    return hp.reshape(N_PAD, P, LANE).transpose(1, 0, 2).reshape(P * N_PAD, LANE)


def kernel(x, edge_index, W1_l, b1, W1_r, W2_l, b2, W2_r, W3_l, b3, W3_r):
    src = edge_index[0].reshape(NS, EPT)
    dst = edge_index[1].reshape(NS, CHUNKS, CHUNK)
    zeros = jnp.zeros((ZR, LANE), jnp.float32)

    deg = _deg_sc()(dst, zeros)[:, 0:1]
    x_pm = _panel_major(x, 2)
    agg1 = _segsum_sc(2)(x_pm, src, dst, zeros)
    h1 = _l1(agg1.reshape(2, N_PAD, LANE), deg,
             x_pm.reshape(2, N_PAD, LANE), W1_l, W1_r, b1.reshape(1, -1))

    agg2 = _segsum_sc(8)(h1.reshape(8 * N_PAD, LANE), src, dst, zeros)
    h2 = _l2(agg2.reshape(8, N_PAD, LANE), deg, h1,
             W2_l, W2_r, b2.reshape(1, -1))

    p3 = _mm3(h2, W3_l)
    agg3 = _segsum_sc(2)(p3.reshape(2 * N_PAD, LANE), src, dst, zeros)
    out = _fin(agg3.reshape(2, N_PAD, LANE), deg, h2, W3_r, b3.reshape(1, -1))
    return out[:N]


# speedup vs baseline: 5.9350x; 1.0786x over previous
import functools

import jax
import jax.numpy as jnp
from jax import lax
from jax.experimental import pallas as pl
from jax.experimental.pallas import tpu as pltpu
from jax.experimental.pallas import tpu_sc as plsc

N = 10000
E = 160000
N_PAD = 10240
LANE = 128
NC, NS = 2, 16
EPT = E // NS
CHUNK = 80
CHUNKS = EPT // CHUNK
ZR = N_PAD // NS


def _sc_mesh():
    return plsc.VectorSubcoreMesh(core_axis_name="c", subcore_axis_name="s",
                                  num_cores=NC, num_subcores=NS)


@functools.cache
def _segsum_sc(P):
    split = (P + 1) // 2
    D = 4
    STEPS = (CHUNKS - 1) // D

    @functools.partial(
        pl.kernel,
        out_type=jax.ShapeDtypeStruct((P * N_PAD, LANE), jnp.float32),
        mesh=_sc_mesh(),
        scratch_types=[
            [pltpu.VMEM((CHUNK,), jnp.int32) for _ in range(D)],
            pltpu.VMEM((D, CHUNK), jnp.int32),
            [pltpu.VMEM((CHUNK, LANE), jnp.float32) for _ in range(D)],
            pltpu.VMEM_SHARED((N_PAD, LANE), jnp.float32),
            [pltpu.SemaphoreType.DMA for _ in range(D)],
            [pltpu.SemaphoreType.DMA for _ in range(D)],
        ],
    )
    def k(table, srcpo, dstfl, zeros, out, gidx, didx, rows, acc, isem, gsem):
        c = lax.axis_index("c")
        s = lax.axis_index("s")
        my_base = c * split
        my_count = jnp.where(c == 0, split, P - split)
        for pp in range(split):
            @pl.when(pp < my_count)
            def _():
                p = my_base + pp
                ibase = (p * NS + s) * EPT
                dbase = s * EPT

                def idx_start(j, b):
                    pltpu.async_copy(
                        srcpo.at[pl.ds(ibase + j * CHUNK, CHUNK)],
                        gidx[b], isem[b])
                    pltpu.async_copy(
                        dstfl.at[pl.ds(dbase + j * CHUNK, CHUNK)],
                        didx.at[b], isem[b])

                def idx_wait(b):
                    pltpu.make_async_copy(
                        srcpo.at[pl.ds(0, CHUNK)], gidx[b], isem[b]).wait()
                    pltpu.make_async_copy(
                        dstfl.at[pl.ds(0, CHUNK)], didx.at[b], isem[b]).wait()

                pltpu.sync_copy(zeros, acc.at[pl.ds(s * ZR, ZR)])
                plsc.subcore_barrier()
                for b in range(D):
                    idx_start(b, b)
                for b in range(D):
                    idx_wait(b)
                    pltpu.async_copy(table.at[gidx[b]], rows[b], gsem[b])

                def step(t, _):
                    for b in range(D):
                        j = D * t + b
                        pltpu.make_async_copy(table.at[gidx[b]], rows[b],
                                              gsem[b]).wait()
                        pltpu.sync_copy(rows[b], acc.at[didx.at[b]], add=True)

                        @pl.when(j + D < CHUNKS)
                        def _():
                            idx_start(j + D, b)
                    for b in range(D):
                        j = D * t + b

                        @pl.when(j + D < CHUNKS)
                        def _():
                            idx_wait(b)
                            pltpu.async_copy(table.at[gidx[b]], rows[b],
                                             gsem[b])
                    return ()

                lax.fori_loop(0, STEPS, step, (), unroll=False)
                pltpu.make_async_copy(table.at[gidx[0]], rows[0],
                                      gsem[0]).wait()
                pltpu.sync_copy(rows[0], acc.at[didx.at[0]], add=True)
                plsc.subcore_barrier()
                pltpu.sync_copy(acc.at[pl.ds(s * ZR, ZR)],
                                out.at[pl.ds((p * N_PAD) + s * ZR, ZR)])
                plsc.subcore_barrier()

    return lambda *a: k(*a)


@functools.cache
def _deg_sc():
    @functools.partial(
        pl.kernel,
        out_type=jax.ShapeDtypeStruct((N_PAD, LANE), jnp.float32),
        mesh=_sc_mesh(),
        scratch_types=[
            pltpu.VMEM((CHUNKS, CHUNK), jnp.int32),
            pltpu.VMEM((CHUNK, LANE), jnp.float32),
            pltpu.VMEM_SHARED((N_PAD, LANE), jnp.float32),
            pltpu.SemaphoreType.DMA,
        ],
    )
    def k(dst, zeros, out, dstb, onesb, dacc, sem):
        c = lax.axis_index("c")
        s = lax.axis_index("s")

        @pl.when(c == 0)
        def _():
            pltpu.sync_copy(dst.at[s], dstb)
            ones16 = jnp.ones((16,), jnp.float32)
            for r in range(CHUNK):
                for w in range(LANE // 16):
                    onesb[r, pl.ds(w * 16, 16)] = ones16
            pltpu.sync_copy(zeros, dacc.at[pl.ds(s * ZR, ZR)])
            plsc.subcore_barrier()

            def fire(j, _):
                pltpu.sync_copy(onesb, dacc.at[dstb.at[j]], add=True)
                return ()

            lax.fori_loop(0, CHUNKS, fire, (), unroll=False)
            plsc.subcore_barrier()
            pltpu.sync_copy(dacc.at[pl.ds(s * ZR, ZR)],
                            out.at[pl.ds(s * ZR, ZR)])

    return k


def _tc_layer(P_in, D_out, BN=256, JB=512, interpret=False):
    JP = JB // LANE
    OP = D_out // LANE
    K = P_in * LANE

    def body(agg_ref, deg_ref, x_ref, wl_ref, wr_ref, b_ref, out_ref):
        invd = 1.0 / jnp.maximum(deg_ref[...], 1.0)
        a = jnp.concatenate([agg_ref[p] for p in range(P_in)], axis=1)
        xx = jnp.concatenate([x_ref[p] for p in range(P_in)], axis=1)
        acc = jnp.dot(a, wl_ref[...], preferred_element_type=jnp.float32) * invd
        acc = acc + jnp.dot(xx, wr_ref[...], preferred_element_type=jnp.float32)
        acc = jnp.maximum(acc + b_ref[...], 0.0)
        for q in range(JP):
            out_ref[q] = acc[:, q * LANE:(q + 1) * LANE]

    return pl.pallas_call(
        body,
        grid=(D_out // JB, N_PAD // BN),
        in_specs=[
            pl.BlockSpec((P_in, BN, LANE), lambda j, n: (0, n, 0)),
            pl.BlockSpec((BN, 1), lambda j, n: (n, 0)),
            pl.BlockSpec((P_in, BN, LANE), lambda j, n: (0, n, 0)),
            pl.BlockSpec((K, JB), lambda j, n: (0, j)),
            pl.BlockSpec((K, JB), lambda j, n: (0, j)),
            pl.BlockSpec((1, JB), lambda j, n: (0, j)),
        ],
        out_specs=pl.BlockSpec((JP, BN, LANE), lambda j, n: (j, n, 0)),
        out_shape=jax.ShapeDtypeStruct((OP, N_PAD, LANE), jnp.float32),
        interpret=interpret,
    )


def _tc_matmul(P_in, D_out, BN=256, interpret=False):
    OP = D_out // LANE
    K = P_in * LANE

    def body(x_ref, w_ref, out_ref):
        xx = jnp.concatenate([x_ref[p] for p in range(P_in)], axis=1)
        acc = jnp.dot(xx, w_ref[...], preferred_element_type=jnp.float32)
        for q in range(OP):
            out_ref[q] = acc[:, q * LANE:(q + 1) * LANE]

    return pl.pallas_call(
        body,
        grid=(N_PAD // BN,),
        in_specs=[
            pl.BlockSpec((P_in, BN, LANE), lambda n: (0, n, 0)),
            pl.BlockSpec((K, D_out), lambda n: (0, 0)),
        ],
        out_specs=pl.BlockSpec((OP, BN, LANE), lambda n: (0, n, 0)),
        out_shape=jax.ShapeDtypeStruct((OP, N_PAD, LANE), jnp.float32),
        interpret=interpret,
    )


def _tc_final(P_in, D_out, BN=256, interpret=False):
    AP = D_out // LANE
    K = P_in * LANE

    def body(agg_ref, deg_ref, x_ref, wr_ref, b_ref, out_ref):
        invd = 1.0 / jnp.maximum(deg_ref[...], 1.0)
        xx = jnp.concatenate([x_ref[p] for p in range(P_in)], axis=1)
        acc = jnp.dot(xx, wr_ref[...], preferred_element_type=jnp.float32)
        agg = jnp.concatenate([agg_ref[q] for q in range(AP)], axis=1)
        out_ref[...] = jnp.maximum(acc + agg * invd + b_ref[...], 0.0)

    return pl.pallas_call(
        body,
        grid=(N_PAD // BN,),
        in_specs=[
            pl.BlockSpec((AP, BN, LANE), lambda n: (0, n, 0)),
            pl.BlockSpec((BN, 1), lambda n: (n, 0)),
            pl.BlockSpec((P_in, BN, LANE), lambda n: (0, n, 0)),
            pl.BlockSpec((K, D_out), lambda n: (0, 0)),
            pl.BlockSpec((1, D_out), lambda n: (0, 0)),
        ],
        out_specs=pl.BlockSpec((BN, D_out), lambda n: (n, 0)),
        out_shape=jax.ShapeDtypeStruct((N_PAD, D_out), jnp.float32),
        interpret=interpret,
    )


_l1 = _tc_layer(2, 1024)
_l2 = _tc_layer(8, 1024)
_mm3 = _tc_matmul(8, 256)
_fin = _tc_final(8, 256)


def _panel_major(h, P):
    hp = jnp.pad(h, ((0, N_PAD - h.shape[0]), (0, 0)))
    return hp.reshape(N_PAD, P, LANE).transpose(1, 0, 2).reshape(P * N_PAD, LANE)


def _src_po(src, P):
    offs = (jnp.arange(P, dtype=jnp.int32) * N_PAD).reshape(P, 1, 1)
    return (src.reshape(1, NS, EPT) + offs).reshape(-1)


def kernel(x, edge_index, W1_l, b1, W1_r, W2_l, b2, W2_r, W3_l, b3, W3_r):
    src = edge_index[0]
    dst3 = edge_index[1].reshape(NS, CHUNKS, CHUNK)
    dstfl = edge_index[1]
    zeros = jnp.zeros((ZR, LANE), jnp.float32)

    deg = _deg_sc()(dst3, zeros)[:, 0:1]
    x_pm = _panel_major(x, 2)
    agg1 = _segsum_sc(2)(x_pm, _src_po(src, 2), dstfl, zeros)
    h1 = _l1(agg1.reshape(2, N_PAD, LANE), deg,
             x_pm.reshape(2, N_PAD, LANE), W1_l, W1_r, b1.reshape(1, -1))

    agg2 = _segsum_sc(8)(h1.reshape(8 * N_PAD, LANE), _src_po(src, 8),
                         dstfl, zeros)
    h2 = _l2(agg2.reshape(8, N_PAD, LANE), deg, h1,
             W2_l, W2_r, b2.reshape(1, -1))

    p3 = _mm3(h2, W3_l)
    agg3 = _segsum_sc(2)(p3.reshape(2 * N_PAD, LANE), _src_po(src, 2),
                         dstfl, zeros)
    out = _fin(agg3.reshape(2, N_PAD, LANE), deg, h2, W3_r, b3.reshape(1, -1))
    return out[:N]


# speedup vs baseline: 6.0405x; 1.0178x over previous
import functools

import jax
import jax.numpy as jnp
from jax import lax
from jax.experimental import pallas as pl
from jax.experimental.pallas import tpu as pltpu
from jax.experimental.pallas import tpu_sc as plsc

N = 10000
E = 160000
N_PAD = 10240
LANE = 128
NC, NS = 2, 16
EPT = E // NS
CHUNK = 80
CHUNKS = EPT // CHUNK
ZR = N_PAD // NS


def _sc_mesh():
    return plsc.VectorSubcoreMesh(core_axis_name="c", subcore_axis_name="s",
                                  num_cores=NC, num_subcores=NS)


@functools.cache
def _segsum_sc(P):
    split = (P + 1) // 2
    D = 4
    STEPS = (CHUNKS - 1) // D

    @functools.partial(
        pl.kernel,
        out_type=jax.ShapeDtypeStruct((P * N_PAD, LANE), jnp.float32),
        mesh=_sc_mesh(),
        scratch_types=[
            [pltpu.VMEM((CHUNK,), jnp.int32) for _ in range(D)],
            pltpu.VMEM((D, CHUNK), jnp.int32),
            [pltpu.VMEM((CHUNK, LANE), jnp.float32) for _ in range(D)],
            pltpu.VMEM_SHARED((N_PAD, LANE), jnp.float32),
            [pltpu.SemaphoreType.DMA for _ in range(D)],
            [pltpu.SemaphoreType.DMA for _ in range(D)],
        ],
    )
    def k(table, srcpo, dstfl, zeros, out, gidx, didx, rows, acc, isem, gsem):
        c = lax.axis_index("c")
        s = lax.axis_index("s")
        my_base = c * split
        my_count = jnp.where(c == 0, split, P - split)
        for pp in range(split):
            @pl.when(pp < my_count)
            def _():
                p = my_base + pp
                ibase = (p * NS + s) * EPT
                dbase = s * EPT

                def idx_start(j, b):
                    pltpu.async_copy(
                        srcpo.at[pl.ds(ibase + j * CHUNK, CHUNK)],
                        gidx[b], isem[b])
                    pltpu.async_copy(
                        dstfl.at[pl.ds(dbase + j * CHUNK, CHUNK)],
                        didx.at[b], isem[b])

                def idx_wait(b):
                    pltpu.make_async_copy(
                        srcpo.at[pl.ds(0, CHUNK)], gidx[b], isem[b]).wait()
                    pltpu.make_async_copy(
                        dstfl.at[pl.ds(0, CHUNK)], didx.at[b], isem[b]).wait()

                pltpu.sync_copy(zeros, acc.at[pl.ds(s * ZR, ZR)])
                plsc.subcore_barrier()
                for b in range(D):
                    idx_start(b, b)
                for b in range(D):
                    idx_wait(b)
                    pltpu.async_copy(table.at[gidx[b]], rows[b], gsem[b])

                def step(t, _):
                    for b in range(D):
                        j = D * t + b
                        pltpu.make_async_copy(table.at[gidx[b]], rows[b],
                                              gsem[b]).wait()
                        pltpu.sync_copy(rows[b], acc.at[didx.at[b]], add=True)

                        @pl.when(j + D < CHUNKS)
                        def _():
                            idx_start(j + D, b)
                    for b in range(D):
                        j = D * t + b

                        @pl.when(j + D < CHUNKS)
                        def _():
                            idx_wait(b)
                            pltpu.async_copy(table.at[gidx[b]], rows[b],
                                             gsem[b])
                    return ()

                lax.fori_loop(0, STEPS, step, (), unroll=False)
                pltpu.make_async_copy(table.at[gidx[0]], rows[0],
                                      gsem[0]).wait()
                pltpu.sync_copy(rows[0], acc.at[didx.at[0]], add=True)
                plsc.subcore_barrier()
                pltpu.sync_copy(acc.at[pl.ds(s * ZR, ZR)],
                                out.at[pl.ds((p * N_PAD) + s * ZR, ZR)])
                plsc.subcore_barrier()

    return lambda *a: k(*a)


@functools.cache
def _deg_sc():
    @functools.partial(
        pl.kernel,
        out_type=jax.ShapeDtypeStruct((N_PAD, LANE), jnp.float32),
        mesh=_sc_mesh(),
        scratch_types=[
            pltpu.VMEM((CHUNKS, CHUNK), jnp.int32),
            pltpu.VMEM((CHUNK, LANE), jnp.float32),
            pltpu.VMEM_SHARED((N_PAD, LANE), jnp.float32),
            pltpu.SemaphoreType.DMA,
        ],
    )
    def k(dst, zeros, out, dstb, onesb, dacc, sem):
        c = lax.axis_index("c")
        s = lax.axis_index("s")

        @pl.when(c == 0)
        def _():
            pltpu.sync_copy(dst.at[s], dstb)
            ones16 = jnp.ones((16,), jnp.float32)
            for r in range(CHUNK):
                for w in range(LANE // 16):
                    onesb[r, pl.ds(w * 16, 16)] = ones16
            pltpu.sync_copy(zeros, dacc.at[pl.ds(s * ZR, ZR)])
            plsc.subcore_barrier()

            def fire(j, _):
                pltpu.sync_copy(onesb, dacc.at[dstb.at[j]], add=True)
                return ()

            lax.fori_loop(0, CHUNKS, fire, (), unroll=False)
            plsc.subcore_barrier()
            pltpu.sync_copy(dacc.at[pl.ds(s * ZR, ZR)],
                            out.at[pl.ds(s * ZR, ZR)])

    return k


def _tc_layer(P_in, D_out, BN=256, JB=512, interpret=False):
    JP = JB // LANE
    OP = D_out // LANE
    K = P_in * LANE

    def body(agg_ref, deg_ref, x_ref, wl_ref, wr_ref, b_ref, out_ref):
        invd = 1.0 / jnp.maximum(deg_ref[...], 1.0)
        a = jnp.concatenate([agg_ref[p] for p in range(P_in)], axis=1)
        xx = jnp.concatenate([x_ref[p] for p in range(P_in)], axis=1)
        acc = jnp.dot(a, wl_ref[...], preferred_element_type=jnp.float32) * invd
        acc = acc + jnp.dot(xx, wr_ref[...], preferred_element_type=jnp.float32)
        acc = jnp.maximum(acc + b_ref[...], 0.0)
        for q in range(JP):
            out_ref[q] = acc[:, q * LANE:(q + 1) * LANE]

    return pl.pallas_call(
        body,
        grid=(D_out // JB, N_PAD // BN),
        in_specs=[
            pl.BlockSpec((P_in, BN, LANE), lambda j, n: (0, n, 0)),
            pl.BlockSpec((BN, 1), lambda j, n: (n, 0)),
            pl.BlockSpec((P_in, BN, LANE), lambda j, n: (0, n, 0)),
            pl.BlockSpec((K, JB), lambda j, n: (0, j)),
            pl.BlockSpec((K, JB), lambda j, n: (0, j)),
            pl.BlockSpec((1, JB), lambda j, n: (0, j)),
        ],
        out_specs=pl.BlockSpec((JP, BN, LANE), lambda j, n: (j, n, 0)),
        out_shape=jax.ShapeDtypeStruct((OP, N_PAD, LANE), jnp.float32),
        interpret=interpret,
    )


def _tc_matmul(P_in, D_out, BN=256, bias=False, interpret=False):
    OP = D_out // LANE
    K = P_in * LANE

    def body(*refs):
        if bias:
            x_ref, w_ref, b_ref, out_ref = refs
        else:
            x_ref, w_ref, out_ref = refs
        xx = jnp.concatenate([x_ref[p] for p in range(P_in)], axis=1)
        acc = jnp.dot(xx, w_ref[...], preferred_element_type=jnp.float32)
        if bias:
            acc = acc + b_ref[...]
        for q in range(OP):
            out_ref[q] = acc[:, q * LANE:(q + 1) * LANE]

    in_specs = [
        pl.BlockSpec((P_in, BN, LANE), lambda n: (0, n, 0)),
        pl.BlockSpec((K, D_out), lambda n: (0, 0)),
    ]
    if bias:
        in_specs.append(pl.BlockSpec((1, D_out), lambda n: (0, 0)))
    return pl.pallas_call(
        body,
        grid=(N_PAD // BN,),
        in_specs=in_specs,
        out_specs=pl.BlockSpec((OP, BN, LANE), lambda n: (0, n, 0)),
        out_shape=jax.ShapeDtypeStruct((OP, N_PAD, LANE), jnp.float32),
        interpret=interpret,
    )


def _tc_left(P_in, D_out, BN=256, JB=512, interpret=False):
    JP = JB // LANE
    OP = D_out // LANE
    K = P_in * LANE

    def body(agg_ref, deg_ref, r_ref, wl_ref, out_ref):
        invd = 1.0 / jnp.maximum(deg_ref[...], 1.0)
        a = jnp.concatenate([agg_ref[p] for p in range(P_in)], axis=1)
        acc = jnp.dot(a, wl_ref[...], preferred_element_type=jnp.float32) * invd
        rr = jnp.concatenate([r_ref[q] for q in range(JP)], axis=1)
        acc = jnp.maximum(acc + rr, 0.0)
        for q in range(JP):
            out_ref[q] = acc[:, q * LANE:(q + 1) * LANE]

    return pl.pallas_call(
        body,
        grid=(D_out // JB, N_PAD // BN),
        in_specs=[
            pl.BlockSpec((P_in, BN, LANE), lambda j, n: (0, n, 0)),
            pl.BlockSpec((BN, 1), lambda j, n: (n, 0)),
            pl.BlockSpec((JP, BN, LANE), lambda j, n: (j, n, 0)),
            pl.BlockSpec((K, JB), lambda j, n: (0, j)),
        ],
        out_specs=pl.BlockSpec((JP, BN, LANE), lambda j, n: (j, n, 0)),
        out_shape=jax.ShapeDtypeStruct((OP, N_PAD, LANE), jnp.float32),
        interpret=interpret,
    )


def _tc_finb(D_out=256, BN=256, interpret=False):
    AP = D_out // LANE

    def body(agg_ref, deg_ref, r_ref, out_ref):
        invd = 1.0 / jnp.maximum(deg_ref[...], 1.0)
        agg = jnp.concatenate([agg_ref[q] for q in range(AP)], axis=1)
        rr = jnp.concatenate([r_ref[q] for q in range(AP)], axis=1)
        out_ref[...] = jnp.maximum(agg * invd + rr, 0.0)

    return pl.pallas_call(
        body,
        grid=(N_PAD // BN,),
        in_specs=[
            pl.BlockSpec((AP, BN, LANE), lambda n: (0, n, 0)),
            pl.BlockSpec((BN, 1), lambda n: (n, 0)),
            pl.BlockSpec((AP, BN, LANE), lambda n: (0, n, 0)),
        ],
        out_specs=pl.BlockSpec((BN, D_out), lambda n: (n, 0)),
        out_shape=jax.ShapeDtypeStruct((N_PAD, D_out), jnp.float32),
        interpret=interpret,
    )


def _tc_final(P_in, D_out, BN=256, interpret=False):
    AP = D_out // LANE
    K = P_in * LANE

    def body(agg_ref, deg_ref, x_ref, wr_ref, b_ref, out_ref):
        invd = 1.0 / jnp.maximum(deg_ref[...], 1.0)
        xx = jnp.concatenate([x_ref[p] for p in range(P_in)], axis=1)
        acc = jnp.dot(xx, wr_ref[...], preferred_element_type=jnp.float32)
        agg = jnp.concatenate([agg_ref[q] for q in range(AP)], axis=1)
        out_ref[...] = jnp.maximum(acc + agg * invd + b_ref[...], 0.0)

    return pl.pallas_call(
        body,
        grid=(N_PAD // BN,),
        in_specs=[
            pl.BlockSpec((AP, BN, LANE), lambda n: (0, n, 0)),
            pl.BlockSpec((BN, 1), lambda n: (n, 0)),
            pl.BlockSpec((P_in, BN, LANE), lambda n: (0, n, 0)),
            pl.BlockSpec((K, D_out), lambda n: (0, 0)),
            pl.BlockSpec((1, D_out), lambda n: (0, 0)),
        ],
        out_specs=pl.BlockSpec((BN, D_out), lambda n: (n, 0)),
        out_shape=jax.ShapeDtypeStruct((N_PAD, D_out), jnp.float32),
        interpret=interpret,
    )


_r1 = _tc_matmul(2, 1024, bias=True)
_r2 = _tc_matmul(8, 1024, bias=True)
_r3 = _tc_matmul(8, 256, bias=True)
_left1 = _tc_left(2, 1024)
_left2 = _tc_left(8, 1024)
_mm3 = _tc_matmul(8, 256)
_finb = _tc_finb(256)


def _panel_major(h, P):
    hp = jnp.pad(h, ((0, N_PAD - h.shape[0]), (0, 0)))
    return hp.reshape(N_PAD, P, LANE).transpose(1, 0, 2).reshape(P * N_PAD, LANE)


def _src_po(src, P):
    offs = (jnp.arange(P, dtype=jnp.int32) * N_PAD).reshape(P, 1, 1)
    return (src.reshape(1, NS, EPT) + offs).reshape(-1)


def kernel(x, edge_index, W1_l, b1, W1_r, W2_l, b2, W2_r, W3_l, b3, W3_r):
    src = edge_index[0]
    dst3 = edge_index[1].reshape(NS, CHUNKS, CHUNK)
    dstfl = edge_index[1]
    zeros = jnp.zeros((ZR, LANE), jnp.float32)

    deg = _deg_sc()(dst3, zeros)[:, 0:1]
    x_pm = _panel_major(x, 2)
    x_p = x_pm.reshape(2, N_PAD, LANE)
    agg1 = _segsum_sc(2)(x_pm, _src_po(src, 2), dstfl, zeros)
    r1 = _r1(x_p, W1_r, b1.reshape(1, -1))
    h1 = _left1(agg1.reshape(2, N_PAD, LANE), deg, r1, W1_l)

    agg2 = _segsum_sc(8)(h1.reshape(8 * N_PAD, LANE), _src_po(src, 8),
                         dstfl, zeros)
    r2 = _r2(h1, W2_r, b2.reshape(1, -1))
    h2 = _left2(agg2.reshape(8, N_PAD, LANE), deg, r2, W2_l)

    p3 = _mm3(h2, W3_l)
    agg3 = _segsum_sc(2)(p3.reshape(2 * N_PAD, LANE), _src_po(src, 2),
                         dstfl, zeros)
    r3 = _r3(h2, W3_r, b3.reshape(1, -1))
    out = _finb(agg3.reshape(2, N_PAD, LANE), deg, r3)
    return out[:N]


# speedup vs baseline: 7.0369x; 1.1649x over previous
import functools

import jax
import jax.numpy as jnp
from jax import lax
from jax.experimental import pallas as pl
from jax.experimental.pallas import tpu as pltpu
from jax.experimental.pallas import tpu_sc as plsc

N = 10000
E = 160000
N_PAD = 10240
LANE = 128
NC, NS = 2, 16
EPT = E // NS
CHUNK = 80
CHUNKS = EPT // CHUNK
ZR = N_PAD // NS


def _sc_mesh():
    return plsc.VectorSubcoreMesh(core_axis_name="c", subcore_axis_name="s",
                                  num_cores=NC, num_subcores=NS)


@functools.cache
def _segsum_sc(P):
    split = (P + 1) // 2
    D = 4
    STEPS = (CHUNKS - 1) // D

    @functools.partial(
        pl.kernel,
        out_type=jax.ShapeDtypeStruct((P * N_PAD, LANE), jnp.float32),
        mesh=_sc_mesh(),
        scratch_types=[
            pltpu.VMEM((2 * D, CHUNK), jnp.int32),
            pltpu.VMEM((2 * D, CHUNK), jnp.int32),
            [pltpu.VMEM((CHUNK, LANE), jnp.float32) for _ in range(D)],
            pltpu.VMEM_SHARED((N_PAD, LANE), jnp.float32),
            [pltpu.SemaphoreType.DMA for _ in range(D)],
            [pltpu.SemaphoreType.DMA for _ in range(D)],
            [pltpu.SemaphoreType.DMA for _ in range(D)],
        ],
    )
    def k(table, srcpo, dstfl, zeros, out, gidx, didx, rows, acc,
          isem, gsem, ssem):
        c = lax.axis_index("c")
        s = lax.axis_index("s")
        my_base = c * split
        my_count = jnp.where(c == 0, split, P - split)
        for pp in range(split):
            @pl.when(pp < my_count)
            def _():
                p = my_base + pp
                ibase = (p * NS + s) * EPT
                dbase = s * EPT

                def idx_start(j, row, b):
                    pltpu.async_copy(
                        srcpo.at[pl.ds(ibase + j * CHUNK, CHUNK)],
                        gidx.at[row], isem[b])
                    pltpu.async_copy(
                        dstfl.at[pl.ds(dbase + j * CHUNK, CHUNK)],
                        didx.at[row], isem[b])

                def idx_wait(b):
                    pltpu.make_async_copy(
                        srcpo.at[pl.ds(0, CHUNK)], gidx.at[0], isem[b]).wait()
                    pltpu.make_async_copy(
                        dstfl.at[pl.ds(0, CHUNK)], didx.at[0], isem[b]).wait()

                def scat_wait(b):
                    pltpu.make_async_copy(rows[b], acc.at[didx.at[0]],
                                          ssem[b]).wait()

                pltpu.sync_copy(zeros, acc.at[pl.ds(s * ZR, ZR)])
                plsc.subcore_barrier()
                for b in range(D):
                    idx_start(b, b, b)
                    idx_start(b + D, b + D, b)
                for b in range(D):
                    idx_wait(b)
                    pltpu.async_copy(table.at[gidx.at[b]], rows[b], gsem[b])

                def step(t, _):
                    tm = t % 2
                    for b in range(D):
                        j = D * t + b
                        row = b + D * tm
                        pltpu.make_async_copy(table.at[gidx.at[0]], rows[b],
                                              gsem[b]).wait()
                        pltpu.async_copy(rows[b], acc.at[didx.at[row]],
                                         ssem[b], add=True)
                    for b in range(D):
                        j = D * t + b
                        row = b + D * tm
                        nrow = b + D * (1 - tm)

                        @pl.when(j + D < CHUNKS)
                        def _():
                            scat_wait(b)

                            @pl.when(j + 2 * D < CHUNKS)
                            def _():
                                idx_start(j + 2 * D, row, b)
                            idx_wait(b)
                            pltpu.async_copy(table.at[gidx.at[nrow]], rows[b],
                                             gsem[b])
                    return ()

                lax.fori_loop(0, STEPS, step, (), unroll=False)
                pltpu.make_async_copy(table.at[gidx.at[0]], rows[0],
                                      gsem[0]).wait()
                pltpu.async_copy(rows[0], acc.at[didx.at[(CHUNKS - 1) % (2 * D)]],
                                 ssem[0], add=True)
                for b in range(1, D):
                    scat_wait(b)
                scat_wait(0)
                plsc.subcore_barrier()
                pltpu.sync_copy(acc.at[pl.ds(s * ZR, ZR)],
                                out.at[pl.ds((p * N_PAD) + s * ZR, ZR)])
                plsc.subcore_barrier()

    return lambda *a: k(*a)


@functools.cache
def _deg_sc():
    @functools.partial(
        pl.kernel,
        out_type=jax.ShapeDtypeStruct((N_PAD, LANE), jnp.float32),
        mesh=_sc_mesh(),
        scratch_types=[
            pltpu.VMEM((CHUNKS, CHUNK), jnp.int32),
            pltpu.VMEM((CHUNK, LANE), jnp.float32),
            pltpu.VMEM_SHARED((N_PAD, LANE), jnp.float32),
            pltpu.SemaphoreType.DMA,
        ],
    )
    def k(dst, zeros, out, dstb, onesb, dacc, sem):
        c = lax.axis_index("c")
        s = lax.axis_index("s")

        @pl.when(c == 0)
        def _():
            pltpu.sync_copy(dst.at[s], dstb)
            ones16 = jnp.ones((16,), jnp.float32)
            for r in range(CHUNK):
                for w in range(LANE // 16):
                    onesb[r, pl.ds(w * 16, 16)] = ones16
            pltpu.sync_copy(zeros, dacc.at[pl.ds(s * ZR, ZR)])
            plsc.subcore_barrier()

            def fire(j, _):
                pltpu.sync_copy(onesb, dacc.at[dstb.at[j]], add=True)
                return ()

            lax.fori_loop(0, CHUNKS, fire, (), unroll=False)
            plsc.subcore_barrier()
            pltpu.sync_copy(dacc.at[pl.ds(s * ZR, ZR)],
                            out.at[pl.ds(s * ZR, ZR)])

    return k


def _tc_layer(P_in, D_out, BN=256, JB=512, interpret=False):
    JP = JB // LANE
    OP = D_out // LANE
    K = P_in * LANE

    def body(agg_ref, deg_ref, x_ref, wl_ref, wr_ref, b_ref, out_ref):
        invd = 1.0 / jnp.maximum(deg_ref[...], 1.0)
        a = jnp.concatenate([agg_ref[p] for p in range(P_in)], axis=1)
        xx = jnp.concatenate([x_ref[p] for p in range(P_in)], axis=1)
        acc = jnp.dot(a, wl_ref[...], preferred_element_type=jnp.float32) * invd
        acc = acc + jnp.dot(xx, wr_ref[...], preferred_element_type=jnp.float32)
        acc = jnp.maximum(acc + b_ref[...], 0.0)
        for q in range(JP):
            out_ref[q] = acc[:, q * LANE:(q + 1) * LANE]

    return pl.pallas_call(
        body,
        grid=(D_out // JB, N_PAD // BN),
        in_specs=[
            pl.BlockSpec((P_in, BN, LANE), lambda j, n: (0, n, 0)),
            pl.BlockSpec((BN, 1), lambda j, n: (n, 0)),
            pl.BlockSpec((P_in, BN, LANE), lambda j, n: (0, n, 0)),
            pl.BlockSpec((K, JB), lambda j, n: (0, j)),
            pl.BlockSpec((K, JB), lambda j, n: (0, j)),
            pl.BlockSpec((1, JB), lambda j, n: (0, j)),
        ],
        out_specs=pl.BlockSpec((JP, BN, LANE), lambda j, n: (j, n, 0)),
        out_shape=jax.ShapeDtypeStruct((OP, N_PAD, LANE), jnp.float32),
        interpret=interpret,
    )


def _tc_matmul(P_in, D_out, BN=256, bias=False, interpret=False):
    OP = D_out // LANE
    K = P_in * LANE

    def body(*refs):
        if bias:
            x_ref, w_ref, b_ref, out_ref = refs
        else:
            x_ref, w_ref, out_ref = refs
        xx = jnp.concatenate([x_ref[p] for p in range(P_in)], axis=1)
        acc = jnp.dot(xx, w_ref[...], preferred_element_type=jnp.float32)
        if bias:
            acc = acc + b_ref[...]
        for q in range(OP):
            out_ref[q] = acc[:, q * LANE:(q + 1) * LANE]

    in_specs = [
        pl.BlockSpec((P_in, BN, LANE), lambda n: (0, n, 0)),
        pl.BlockSpec((K, D_out), lambda n: (0, 0)),
    ]
    if bias:
        in_specs.append(pl.BlockSpec((1, D_out), lambda n: (0, 0)))
    return pl.pallas_call(
        body,
        grid=(N_PAD // BN,),
        in_specs=in_specs,
        out_specs=pl.BlockSpec((OP, BN, LANE), lambda n: (0, n, 0)),
        out_shape=jax.ShapeDtypeStruct((OP, N_PAD, LANE), jnp.float32),
        interpret=interpret,
    )


def _tc_left(P_in, D_out, BN=256, JB=512, interpret=False):
    JP = JB // LANE
    OP = D_out // LANE
    K = P_in * LANE

    def body(agg_ref, deg_ref, r_ref, wl_ref, out_ref):
        invd = 1.0 / jnp.maximum(deg_ref[...], 1.0)
        a = jnp.concatenate([agg_ref[p] for p in range(P_in)], axis=1)
        acc = jnp.dot(a, wl_ref[...], preferred_element_type=jnp.float32) * invd
        rr = jnp.concatenate([r_ref[q] for q in range(JP)], axis=1)
        acc = jnp.maximum(acc + rr, 0.0)
        for q in range(JP):
            out_ref[q] = acc[:, q * LANE:(q + 1) * LANE]

    return pl.pallas_call(
        body,
        grid=(D_out // JB, N_PAD // BN),
        in_specs=[
            pl.BlockSpec((P_in, BN, LANE), lambda j, n: (0, n, 0)),
            pl.BlockSpec((BN, 1), lambda j, n: (n, 0)),
            pl.BlockSpec((JP, BN, LANE), lambda j, n: (j, n, 0)),
            pl.BlockSpec((K, JB), lambda j, n: (0, j)),
        ],
        out_specs=pl.BlockSpec((JP, BN, LANE), lambda j, n: (j, n, 0)),
        out_shape=jax.ShapeDtypeStruct((OP, N_PAD, LANE), jnp.float32),
        interpret=interpret,
    )


def _tc_finb(D_out=256, BN=256, interpret=False):
    AP = D_out // LANE

    def body(agg_ref, deg_ref, r_ref, out_ref):
        invd = 1.0 / jnp.maximum(deg_ref[...], 1.0)
        agg = jnp.concatenate([agg_ref[q] for q in range(AP)], axis=1)
        rr = jnp.concatenate([r_ref[q] for q in range(AP)], axis=1)
        out_ref[...] = jnp.maximum(agg * invd + rr, 0.0)

    return pl.pallas_call(
        body,
        grid=(N_PAD // BN,),
        in_specs=[
            pl.BlockSpec((AP, BN, LANE), lambda n: (0, n, 0)),
            pl.BlockSpec((BN, 1), lambda n: (n, 0)),
            pl.BlockSpec((AP, BN, LANE), lambda n: (0, n, 0)),
        ],
        out_specs=pl.BlockSpec((BN, D_out), lambda n: (n, 0)),
        out_shape=jax.ShapeDtypeStruct((N_PAD, D_out), jnp.float32),
        interpret=interpret,
    )


def _tc_final(P_in, D_out, BN=256, interpret=False):
    AP = D_out // LANE
    K = P_in * LANE

    def body(agg_ref, deg_ref, x_ref, wr_ref, b_ref, out_ref):
        invd = 1.0 / jnp.maximum(deg_ref[...], 1.0)
        xx = jnp.concatenate([x_ref[p] for p in range(P_in)], axis=1)
        acc = jnp.dot(xx, wr_ref[...], preferred_element_type=jnp.float32)
        agg = jnp.concatenate([agg_ref[q] for q in range(AP)], axis=1)
        out_ref[...] = jnp.maximum(acc + agg * invd + b_ref[...], 0.0)

    return pl.pallas_call(
        body,
        grid=(N_PAD // BN,),
        in_specs=[
            pl.BlockSpec((AP, BN, LANE), lambda n: (0, n, 0)),
            pl.BlockSpec((BN, 1), lambda n: (n, 0)),
            pl.BlockSpec((P_in, BN, LANE), lambda n: (0, n, 0)),
            pl.BlockSpec((K, D_out), lambda n: (0, 0)),
            pl.BlockSpec((1, D_out), lambda n: (0, 0)),
        ],
        out_specs=pl.BlockSpec((BN, D_out), lambda n: (n, 0)),
        out_shape=jax.ShapeDtypeStruct((N_PAD, D_out), jnp.float32),
        interpret=interpret,
    )


_r1 = _tc_matmul(2, 1024, bias=True)
_r2 = _tc_matmul(8, 1024, bias=True)
_r3 = _tc_matmul(8, 256, bias=True)
_left1 = _tc_left(2, 1024)
_left2 = _tc_left(8, 1024)
_mm3 = _tc_matmul(8, 256)
_finb = _tc_finb(256)


def _panel_major(h, P):
    hp = jnp.pad(h, ((0, N_PAD - h.shape[0]), (0, 0)))
    return hp.reshape(N_PAD, P, LANE).transpose(1, 0, 2).reshape(P * N_PAD, LANE)


def _src_po(src, P):
    offs = (jnp.arange(P, dtype=jnp.int32) * N_PAD).reshape(P, 1, 1)
    return (src.reshape(1, NS, EPT) + offs).reshape(-1)


def kernel(x, edge_index, W1_l, b1, W1_r, W2_l, b2, W2_r, W3_l, b3, W3_r):
    src = edge_index[0]
    dst3 = edge_index[1].reshape(NS, CHUNKS, CHUNK)
    dstfl = edge_index[1]
    zeros = jnp.zeros((ZR, LANE), jnp.float32)

    deg = _deg_sc()(dst3, zeros)[:, 0:1]
    x_pm = _panel_major(x, 2)
    x_p = x_pm.reshape(2, N_PAD, LANE)
    agg1 = _segsum_sc(2)(x_pm, _src_po(src, 2), dstfl, zeros)
    r1 = _r1(x_p, W1_r, b1.reshape(1, -1))
    h1 = _left1(agg1.reshape(2, N_PAD, LANE), deg, r1, W1_l)

    agg2 = _segsum_sc(8)(h1.reshape(8 * N_PAD, LANE), _src_po(src, 8),
                         dstfl, zeros)
    r2 = _r2(h1, W2_r, b2.reshape(1, -1))
    h2 = _left2(agg2.reshape(8, N_PAD, LANE), deg, r2, W2_l)

    p3 = _mm3(h2, W3_l)
    agg3 = _segsum_sc(2)(p3.reshape(2 * N_PAD, LANE), _src_po(src, 2),
                         dstfl, zeros)
    r3 = _r3(h2, W3_r, b3.reshape(1, -1))
    out = _finb(agg3.reshape(2, N_PAD, LANE), deg, r3)
    return out[:N]


# speedup vs baseline: 7.0557x; 1.0027x over previous
import functools

import jax
import jax.numpy as jnp
from jax import lax
from jax.experimental import pallas as pl
from jax.experimental.pallas import tpu as pltpu
from jax.experimental.pallas import tpu_sc as plsc

N = 10000
E = 160000
N_PAD = 10240
LANE = 128
NC, NS = 2, 16
EPT = E // NS
CHUNK = 80
CHUNKS = EPT // CHUNK
ZR = N_PAD // NS


def _sc_mesh():
    return plsc.VectorSubcoreMesh(core_axis_name="c", subcore_axis_name="s",
                                  num_cores=NC, num_subcores=NS)


@functools.cache
def _segsum_sc(P):
    split = (P + 1) // 2
    D = 4
    STEPS = (CHUNKS - 1) // D

    @functools.partial(
        pl.kernel,
        out_type=jax.ShapeDtypeStruct((P * N_PAD, LANE), jnp.float32),
        mesh=_sc_mesh(),
        scratch_types=[
            pltpu.VMEM((2 * D, CHUNK), jnp.int32),
            pltpu.VMEM((2 * D, CHUNK), jnp.int32),
            [pltpu.VMEM((CHUNK, LANE), jnp.float32) for _ in range(D)],
            pltpu.VMEM_SHARED((N_PAD, LANE), jnp.float32),
            [pltpu.SemaphoreType.DMA for _ in range(D)],
            [pltpu.SemaphoreType.DMA for _ in range(D)],
            [pltpu.SemaphoreType.DMA for _ in range(D)],
        ],
    )
    def k(table, srcpo, dstfl, zeros, out, gidx, didx, rows, acc,
          isem, gsem, ssem):
        c = lax.axis_index("c")
        s = lax.axis_index("s")
        my_base = c * split
        my_count = jnp.where(c == 0, split, P - split)
        for pp in range(split):
            @pl.when(pp < my_count)
            def _():
                p = my_base + pp
                ibase = (p * NS + s) * EPT
                dbase = s * EPT

                def idx_start(j, row, b):
                    pltpu.async_copy(
                        srcpo.at[pl.ds(ibase + j * CHUNK, CHUNK)],
                        gidx.at[row], isem[b])
                    pltpu.async_copy(
                        dstfl.at[pl.ds(dbase + j * CHUNK, CHUNK)],
                        didx.at[row], isem[b])

                def idx_wait(b):
                    pltpu.make_async_copy(
                        srcpo.at[pl.ds(0, CHUNK)], gidx.at[0], isem[b]).wait()
                    pltpu.make_async_copy(
                        dstfl.at[pl.ds(0, CHUNK)], didx.at[0], isem[b]).wait()

                def scat_wait(b):
                    pltpu.make_async_copy(rows[b], acc.at[didx.at[0]],
                                          ssem[b]).wait()

                pltpu.sync_copy(zeros, acc.at[pl.ds(s * ZR, ZR)])
                plsc.subcore_barrier()
                for b in range(D):
                    idx_start(b, b, b)
                    idx_start(b + D, b + D, b)
                for b in range(D):
                    idx_wait(b)
                    pltpu.async_copy(table.at[gidx.at[b]], rows[b], gsem[b])

                def step(t, _):
                    tm = t % 2
                    for b in range(D):
                        j = D * t + b
                        row = b + D * tm
                        pltpu.make_async_copy(table.at[gidx.at[0]], rows[b],
                                              gsem[b]).wait()
                        pltpu.async_copy(rows[b], acc.at[didx.at[row]],
                                         ssem[b], add=True)
                    for b in range(D):
                        j = D * t + b
                        row = b + D * tm
                        nrow = b + D * (1 - tm)

                        @pl.when(j + D < CHUNKS)
                        def _():
                            scat_wait(b)

                            @pl.when(j + 2 * D < CHUNKS)
                            def _():
                                idx_start(j + 2 * D, row, b)
                            idx_wait(b)
                            pltpu.async_copy(table.at[gidx.at[nrow]], rows[b],
                                             gsem[b])
                    return ()

                lax.fori_loop(0, STEPS, step, (), unroll=False)
                pltpu.make_async_copy(table.at[gidx.at[0]], rows[0],
                                      gsem[0]).wait()
                pltpu.async_copy(rows[0], acc.at[didx.at[(CHUNKS - 1) % (2 * D)]],
                                 ssem[0], add=True)
                for b in range(1, D):
                    scat_wait(b)
                scat_wait(0)
                plsc.subcore_barrier()
                pltpu.sync_copy(acc.at[pl.ds(s * ZR, ZR)],
                                out.at[pl.ds((p * N_PAD) + s * ZR, ZR)])
                plsc.subcore_barrier()

    return lambda *a: k(*a)


@functools.cache
def _deg_sc():
    @functools.partial(
        pl.kernel,
        out_type=jax.ShapeDtypeStruct((N_PAD, LANE), jnp.float32),
        mesh=_sc_mesh(),
        scratch_types=[
            pltpu.VMEM((CHUNKS, CHUNK), jnp.int32),
            pltpu.VMEM((CHUNK, LANE), jnp.float32),
            pltpu.VMEM_SHARED((N_PAD, LANE), jnp.float32),
            pltpu.SemaphoreType.DMA,
        ],
    )
    def k(dst, zeros, out, dstb, onesb, dacc, sem):
        c = lax.axis_index("c")
        s = lax.axis_index("s")

        @pl.when(c == 0)
        def _():
            pltpu.sync_copy(dst.at[s], dstb)
            ones16 = jnp.ones((16,), jnp.float32)
            for r in range(CHUNK):
                for w in range(LANE // 16):
                    onesb[r, pl.ds(w * 16, 16)] = ones16
            pltpu.sync_copy(zeros, dacc.at[pl.ds(s * ZR, ZR)])
            plsc.subcore_barrier()

            def fire(j, _):
                pltpu.async_copy(onesb, dacc.at[dstb.at[j]], sem, add=True)
                return ()

            lax.fori_loop(0, CHUNKS, fire, (), unroll=False)

            def drain(j, _):
                pltpu.make_async_copy(onesb, dacc.at[dstb.at[0]], sem).wait()
                return ()

            lax.fori_loop(0, CHUNKS, drain, (), unroll=False)
            plsc.subcore_barrier()
            pltpu.sync_copy(dacc.at[pl.ds(s * ZR, ZR)],
                            out.at[pl.ds(s * ZR, ZR)])

    return k


def _tc_layer(P_in, D_out, BN=256, JB=512, interpret=False):
    JP = JB // LANE
    OP = D_out // LANE
    K = P_in * LANE

    def body(agg_ref, deg_ref, x_ref, wl_ref, wr_ref, b_ref, out_ref):
        invd = 1.0 / jnp.maximum(deg_ref[...], 1.0)
        a = jnp.concatenate([agg_ref[p] for p in range(P_in)], axis=1)
        xx = jnp.concatenate([x_ref[p] for p in range(P_in)], axis=1)
        acc = jnp.dot(a, wl_ref[...], preferred_element_type=jnp.float32) * invd
        acc = acc + jnp.dot(xx, wr_ref[...], preferred_element_type=jnp.float32)
        acc = jnp.maximum(acc + b_ref[...], 0.0)
        for q in range(JP):
            out_ref[q] = acc[:, q * LANE:(q + 1) * LANE]

    return pl.pallas_call(
        body,
        grid=(D_out // JB, N_PAD // BN),
        in_specs=[
            pl.BlockSpec((P_in, BN, LANE), lambda j, n: (0, n, 0)),
            pl.BlockSpec((BN, 1), lambda j, n: (n, 0)),
            pl.BlockSpec((P_in, BN, LANE), lambda j, n: (0, n, 0)),
            pl.BlockSpec((K, JB), lambda j, n: (0, j)),
            pl.BlockSpec((K, JB), lambda j, n: (0, j)),
            pl.BlockSpec((1, JB), lambda j, n: (0, j)),
        ],
        out_specs=pl.BlockSpec((JP, BN, LANE), lambda j, n: (j, n, 0)),
        out_shape=jax.ShapeDtypeStruct((OP, N_PAD, LANE), jnp.float32),
        interpret=interpret,
    )


def _tc_matmul(P_in, D_out, BN=256, bias=False, interpret=False):
    OP = D_out // LANE
    K = P_in * LANE

    def body(*refs):
        if bias:
            x_ref, w_ref, b_ref, out_ref = refs
        else:
            x_ref, w_ref, out_ref = refs
        xx = jnp.concatenate([x_ref[p] for p in range(P_in)], axis=1)
        acc = jnp.dot(xx.astype(jnp.bfloat16),
                      w_ref[...].astype(jnp.bfloat16),
                      preferred_element_type=jnp.float32)
        if bias:
            acc = acc + b_ref[...]
        for q in range(OP):
            out_ref[q] = acc[:, q * LANE:(q + 1) * LANE]

    in_specs = [
        pl.BlockSpec((P_in, BN, LANE), lambda n: (0, n, 0)),
        pl.BlockSpec((K, D_out), lambda n: (0, 0)),
    ]
    if bias:
        in_specs.append(pl.BlockSpec((1, D_out), lambda n: (0, 0)))
    return pl.pallas_call(
        body,
        grid=(N_PAD // BN,),
        in_specs=in_specs,
        out_specs=pl.BlockSpec((OP, BN, LANE), lambda n: (0, n, 0)),
        out_shape=jax.ShapeDtypeStruct((OP, N_PAD, LANE), jnp.float32),
        interpret=interpret,
    )


def _tc_left(P_in, D_out, BN=256, JB=512, interpret=False):
    JP = JB // LANE
    OP = D_out // LANE
    K = P_in * LANE

    def body(agg_ref, deg_ref, r_ref, wl_ref, out_ref):
        invd = 1.0 / jnp.maximum(deg_ref[...], 1.0)
        a = jnp.concatenate([agg_ref[p] for p in range(P_in)], axis=1)
        acc = jnp.dot(a.astype(jnp.bfloat16),
                      wl_ref[...].astype(jnp.bfloat16),
                      preferred_element_type=jnp.float32) * invd
        rr = jnp.concatenate([r_ref[q] for q in range(JP)], axis=1)
        acc = jnp.maximum(acc + rr, 0.0)
        for q in range(JP):
            out_ref[q] = acc[:, q * LANE:(q + 1) * LANE]

    return pl.pallas_call(
        body,
        grid=(D_out // JB, N_PAD // BN),
        in_specs=[
            pl.BlockSpec((P_in, BN, LANE), lambda j, n: (0, n, 0)),
            pl.BlockSpec((BN, 1), lambda j, n: (n, 0)),
            pl.BlockSpec((JP, BN, LANE), lambda j, n: (j, n, 0)),
            pl.BlockSpec((K, JB), lambda j, n: (0, j)),
        ],
        out_specs=pl.BlockSpec((JP, BN, LANE), lambda j, n: (j, n, 0)),
        out_shape=jax.ShapeDtypeStruct((OP, N_PAD, LANE), jnp.float32),
        interpret=interpret,
    )


def _tc_finb(D_out=256, BN=256, interpret=False):
    AP = D_out // LANE

    def body(agg_ref, deg_ref, r_ref, out_ref):
        invd = 1.0 / jnp.maximum(deg_ref[...], 1.0)
        agg = jnp.concatenate([agg_ref[q] for q in range(AP)], axis=1)
        rr = jnp.concatenate([r_ref[q] for q in range(AP)], axis=1)
        out_ref[...] = jnp.maximum(agg * invd + rr, 0.0)

    return pl.pallas_call(
        body,
        grid=(N_PAD // BN,),
        in_specs=[
            pl.BlockSpec((AP, BN, LANE), lambda n: (0, n, 0)),
            pl.BlockSpec((BN, 1), lambda n: (n, 0)),
            pl.BlockSpec((AP, BN, LANE), lambda n: (0, n, 0)),
        ],
        out_specs=pl.BlockSpec((BN, D_out), lambda n: (n, 0)),
        out_shape=jax.ShapeDtypeStruct((N_PAD, D_out), jnp.float32),
        interpret=interpret,
    )


def _tc_final(P_in, D_out, BN=256, interpret=False):
    AP = D_out // LANE
    K = P_in * LANE

    def body(agg_ref, deg_ref, x_ref, wr_ref, b_ref, out_ref):
        invd = 1.0 / jnp.maximum(deg_ref[...], 1.0)
        xx = jnp.concatenate([x_ref[p] for p in range(P_in)], axis=1)
        acc = jnp.dot(xx, wr_ref[...], preferred_element_type=jnp.float32)
        agg = jnp.concatenate([agg_ref[q] for q in range(AP)], axis=1)
        out_ref[...] = jnp.maximum(acc + agg * invd + b_ref[...], 0.0)

    return pl.pallas_call(
        body,
        grid=(N_PAD // BN,),
        in_specs=[
            pl.BlockSpec((AP, BN, LANE), lambda n: (0, n, 0)),
            pl.BlockSpec((BN, 1), lambda n: (n, 0)),
            pl.BlockSpec((P_in, BN, LANE), lambda n: (0, n, 0)),
            pl.BlockSpec((K, D_out), lambda n: (0, 0)),
            pl.BlockSpec((1, D_out), lambda n: (0, 0)),
        ],
        out_specs=pl.BlockSpec((BN, D_out), lambda n: (n, 0)),
        out_shape=jax.ShapeDtypeStruct((N_PAD, D_out), jnp.float32),
        interpret=interpret,
    )


_r1 = _tc_matmul(2, 1024, bias=True)
_r2 = _tc_matmul(8, 1024, bias=True)
_r3 = _tc_matmul(8, 256, bias=True)
_left1 = _tc_left(2, 1024)
_left2 = _tc_left(8, 1024)
_mm3 = _tc_matmul(8, 256)
_finb = _tc_finb(256)


def _panel_major(h, P):
    hp = jnp.pad(h, ((0, N_PAD - h.shape[0]), (0, 0)))
    return hp.reshape(N_PAD, P, LANE).transpose(1, 0, 2).reshape(P * N_PAD, LANE)


def _src_po(src, P):
    offs = (jnp.arange(P, dtype=jnp.int32) * N_PAD).reshape(P, 1, 1)
    return (src.reshape(1, NS, EPT) + offs).reshape(-1)


def kernel(x, edge_index, W1_l, b1, W1_r, W2_l, b2, W2_r, W3_l, b3, W3_r):
    src = edge_index[0]
    dst3 = edge_index[1].reshape(NS, CHUNKS, CHUNK)
    dstfl = edge_index[1]
    zeros = jnp.zeros((ZR, LANE), jnp.float32)

    deg = _deg_sc()(dst3, zeros)[:, 0:1]
    x_pm = _panel_major(x, 2)
    x_p = x_pm.reshape(2, N_PAD, LANE)
    agg1 = _segsum_sc(2)(x_pm, _src_po(src, 2), dstfl, zeros)
    r1 = _r1(x_p, W1_r, b1.reshape(1, -1))
    h1 = _left1(agg1.reshape(2, N_PAD, LANE), deg, r1, W1_l)

    agg2 = _segsum_sc(8)(h1.reshape(8 * N_PAD, LANE), _src_po(src, 8),
                         dstfl, zeros)
    r2 = _r2(h1, W2_r, b2.reshape(1, -1))
    h2 = _left2(agg2.reshape(8, N_PAD, LANE), deg, r2, W2_l)

    p3 = _mm3(h2, W3_l)
    agg3 = _segsum_sc(2)(p3.reshape(2 * N_PAD, LANE), _src_po(src, 2),
                         dstfl, zeros)
    r3 = _r3(h2, W3_r, b3.reshape(1, -1))
    out = _finb(agg3.reshape(2, N_PAD, LANE), deg, r3)
    return out[:N]
